# baseline (speedup 1.0000x reference)
import functools

import jax
import jax.numpy as jnp
from jax import lax
from jax.experimental import pallas as pl
from jax.experimental.pallas import tpu as pltpu
from jax.experimental.pallas import tpu_sc as plsc

N_NODES = 10000
N_EDGES = 320000
D_FEAT = 128
HIDDEN = 128
N_CLASSES = 3
D2 = 128

NC, NS, L = 2, 16, 16
NW = NC * NS
NPAD = 10240
CHUNK = 80
NCHUNKS = N_EDGES // CHUNK
CPW = -(-NCHUNKS // NW)
ROWS_PT = NPAD // NS
BPT = ROWS_PT // CHUNK
EPW = N_EDGES // NW
TPW = EPW // CHUNK
SCHUNK = 128
STPW = EPW // SCHUNK
STAIL = EPW - STPW * SCHUNK
SNB = 2
SNG = STPW // SNB
SBPT = ROWS_PT // SCHUNK

_MESH = plsc.VectorSubcoreMesh(core_axis_name="c", subcore_axis_name="s")


def _zero_buf(buf, d, n=None):
    n = CHUNK if n is None else n
    zv = jnp.zeros((L,), jnp.float32)
    if d == 0:
        def body(i, _):
            buf[pl.ds(i * L, L)] = zv
            return 0
        lax.fori_loop(0, n // L, body, 0)
    else:
        def body(i, _):
            for j in range(d // L):
                buf[i, pl.ds(j * L, L)] = zv
            return 0
        lax.fori_loop(0, n, body, 0)


DNB = 4
DEPW = N_EDGES // NW // CHUNK
DNG = DEPW // DNB
DNREM = DEPW - DNG * DNB


@functools.partial(
    pl.kernel,
    out_type=jax.ShapeDtypeStruct((NC * NPAD,), jnp.float32),
    mesh=_MESH,
    scratch_types=[
        pltpu.VMEM((DNB, CHUNK), jnp.int32),
        pltpu.VMEM((CHUNK,), jnp.float32),
        pltpu.VMEM((ROWS_PT,), jnp.float32),
        pltpu.VMEM_SHARED((NPAD,), jnp.float32),
        pltpu.SemaphoreType.DMA,
        pltpu.SemaphoreType.DMA,
        pltpu.SemaphoreType.DMA,
        pltpu.SemaphoreType.DMA,
        pltpu.SemaphoreType.DMA,
        pltpu.SemaphoreType.DMA,
        pltpu.SemaphoreType.DMA,
        pltpu.SemaphoreType.DMA,
    ],
)
def _deg_kernel(dst_hbm, out_hbm, didx, buf, wb, acc,
                si0, si1, si2, si3, ss0, ss1, ss2, ss3):
    c = lax.axis_index("c")
    s = lax.axis_index("s")
    wid = s * NC + c
    base = wid * EPW
    sem_i = [si0, si1, si2, si3]
    sem_s = [ss0, ss1, ss2, ss3]
    _zero_buf(wb, 0, ROWS_PT)
    pltpu.sync_copy(wb, acc.at[pl.ds(s * ROWS_PT, ROWS_PT)])
    plsc.subcore_barrier()
    ov = jnp.ones((L,), jnp.float32)

    def fill(i, _):
        buf[pl.ds(i * L, L)] = ov
        return 0
    lax.fori_loop(0, CHUNK // L, fill, 0)

    def outer(i, _):
        ihandles = []
        for b in range(DNB):
            off = base + (i * DNB + b) * CHUNK

            @pl.when(i > 0)
            def _():
                pltpu.make_async_copy(dst_hbm.at[pl.ds(0, CHUNK)],
                                      didx.at[b], sem_s[b]).wait()
            ihandles.append(
                pltpu.async_copy(dst_hbm.at[pl.ds(off, CHUNK)],
                                 didx.at[b], sem_i[b]))
        for b in range(DNB):
            ihandles[b].wait()
            pltpu.async_copy(buf, acc.at[didx.at[b]], sem_s[b], add=True)
        return 0
    lax.fori_loop(0, DNG, outer, 0)
    for b in range(DNB):
        pltpu.make_async_copy(dst_hbm.at[pl.ds(0, CHUNK)], didx.at[b],
                              sem_s[b]).wait()
    for r in range(DNREM):
        off = base + (DNG * DNB + r) * CHUNK
        pltpu.sync_copy(dst_hbm.at[pl.ds(off, CHUNK)], didx.at[0])
        pltpu.sync_copy(buf, acc.at[didx.at[0]], add=True)
    plsc.subcore_barrier()
    pltpu.sync_copy(acc.at[pl.ds(s * ROWS_PT, ROWS_PT)], wb)
    pltpu.sync_copy(wb, out_hbm.at[pl.ds(c * NPAD + s * ROWS_PT, ROWS_PT)])


def _make_spmm(d):
    @functools.partial(
        pl.kernel,
        out_type=jax.ShapeDtypeStruct((NC, NPAD, d), jnp.float32),
        mesh=_MESH,
        scratch_types=[
            pltpu.VMEM((EPW,), jnp.int32),
            pltpu.VMEM((SNB, SCHUNK), jnp.int32),
            pltpu.VMEM((SNB, SCHUNK, d), jnp.float32),
            pltpu.VMEM((STAIL,), jnp.int32),
            pltpu.VMEM_SHARED((NPAD, d), jnp.float32),
            pltpu.SemaphoreType.DMA,
            pltpu.SemaphoreType.DMA,
            pltpu.SemaphoreType.DMA,
            pltpu.SemaphoreType.DMA,
            pltpu.SemaphoreType.DMA,
            pltpu.SemaphoreType.DMA,
            pltpu.SemaphoreType.DMA,
            pltpu.SemaphoreType.DMA,
            pltpu.SemaphoreType.DMA,
        ],
    )
    def spmm(src_hbm, dst_hbm, h_hbm, out_hbm, sidx_all, didx, rows, tdidx,
             acc, si0, si1, si2, sg0, sg1, sg2, ss0, ss1, ss2):
        c = lax.axis_index("c")
        s = lax.axis_index("s")
        wid = s * NC + c
        base = wid * EPW
        sem_i = [si0, si1, si2]
        sem_g = [sg0, sg1, sg2]
        sem_s = [ss0, ss1, ss2]

        pre = pltpu.async_copy(src_hbm.at[pl.ds(base, EPW)], sidx_all, si0)

        zv = jnp.zeros((L,), jnp.float32)

        def zbody(i, _):
            for j in range(d // L):
                rows[0, i, pl.ds(j * L, L)] = zv
            return 0
        lax.fori_loop(0, SCHUNK, zbody, 0)
        for k in range(SBPT):
            pltpu.sync_copy(rows.at[0],
                            acc.at[pl.ds(s * ROWS_PT + k * SCHUNK, SCHUNK)])
        pre.wait()
        plsc.subcore_barrier()

        def outer(i, _):
            ihandles = []
            ghandles = []
            for b in range(SNB):
                t = i * SNB + b

                @pl.when(i > 0)
                def _():
                    pltpu.make_async_copy(h_hbm.at[pl.ds(0, SCHUNK)],
                                          rows.at[b], sem_s[b]).wait()
                ghandles.append(
                    pltpu.async_copy(
                        h_hbm.at[sidx_all.at[pl.ds(t * SCHUNK, SCHUNK)]],
                        rows.at[b], sem_g[b]))
                ihandles.append(
                    pltpu.async_copy(dst_hbm.at[pl.ds(base + t * SCHUNK, SCHUNK)],
                                     didx.at[b], sem_i[b]))
            for b in range(SNB):
                ghandles[b].wait()
                ihandles[b].wait()
                pltpu.async_copy(rows.at[b], acc.at[didx.at[b]], sem_s[b],
                                 add=True)
            return 0
        lax.fori_loop(0, SNG, outer, 0)
        for b in range(SNB):
            pltpu.make_async_copy(h_hbm.at[pl.ds(0, SCHUNK)], rows.at[b],
                                  sem_s[b]).wait()
        toff = base + STPW * SCHUNK
        pltpu.sync_copy(dst_hbm.at[pl.ds(toff, STAIL)], tdidx)
        pltpu.async_copy(
            h_hbm.at[sidx_all.at[pl.ds(STPW * SCHUNK, STAIL)]],
            rows.at[0].at[pl.ds(0, STAIL)], sg0).wait()
        pltpu.sync_copy(rows.at[0].at[pl.ds(0, STAIL)], acc.at[tdidx], add=True)
        plsc.subcore_barrier()
        for k in range(SBPT):
            r0 = s * ROWS_PT + k * SCHUNK
            pltpu.sync_copy(acc.at[pl.ds(r0, SCHUNK)], rows.at[0])
            pltpu.sync_copy(rows.at[0], out_hbm.at[c, pl.ds(r0, SCHUNK)])

    return spmm


_spmm128 = _make_spmm(HIDDEN)
_spmm2 = _make_spmm(D2)

BR = 640
GRID = NPAD // BR


def _dis(degp_blk):
    d = lax.rsqrt(degp_blk[0] + degp_blk[1] + 1.0)
    return d[:, None]


def _tca_body(x_ref, w1_ref, degp_ref, ht_ref):
    dis = _dis(degp_ref[...])
    h = jnp.dot(x_ref[...], w1_ref[...], preferred_element_type=jnp.float32)
    ht_ref[...] = h * dis


_tca = pl.pallas_call(
    _tca_body,
    grid=(GRID,),
    in_specs=[
        pl.BlockSpec((BR, D_FEAT), lambda i: (i, 0)),
        pl.BlockSpec((D_FEAT, HIDDEN), lambda i: (0, 0)),
        pl.BlockSpec((NC, BR), lambda i: (0, i)),
    ],
    out_specs=pl.BlockSpec((BR, HIDDEN), lambda i: (i, 0)),
    out_shape=jax.ShapeDtypeStruct((N_NODES, HIDDEN), jnp.float32),
)


def _tcb_body(y1_ref, ht1_ref, degp_ref, b1_ref, w2_ref, ht2_ref):
    dis = _dis(degp_ref[...])
    y = y1_ref[0] + y1_ref[1] + ht1_ref[...]
    o1 = y * dis + b1_ref[...]
    r = jnp.maximum(o1, 0.0)
    h2 = jnp.dot(r, w2_ref[...], preferred_element_type=jnp.float32)
    ht2_ref[...] = h2 * dis


_tcb = pl.pallas_call(
    _tcb_body,
    grid=(GRID,),
    in_specs=[
        pl.BlockSpec((NC, BR, HIDDEN), lambda i: (0, i, 0)),
        pl.BlockSpec((BR, HIDDEN), lambda i: (i, 0)),
        pl.BlockSpec((NC, BR), lambda i: (0, i)),
        pl.BlockSpec((1, HIDDEN), lambda i: (0, 0)),
        pl.BlockSpec((HIDDEN, D2), lambda i: (0, 0)),
    ],
    out_specs=pl.BlockSpec((BR, D2), lambda i: (i, 0)),
    out_shape=jax.ShapeDtypeStruct((N_NODES, D2), jnp.float32),
)


def _tcc_body(y2_ref, ht2_ref, degp_ref, b2_ref, out_ref):
    dis = _dis(degp_ref[...])
    y = y2_ref[0] + y2_ref[1] + ht2_ref[...]
    o = y * dis
    out_ref[...] = o[:, :N_CLASSES] + b2_ref[...]


_tcc = pl.pallas_call(
    _tcc_body,
    grid=(GRID,),
    in_specs=[
        pl.BlockSpec((NC, BR, D2), lambda i: (0, i, 0)),
        pl.BlockSpec((BR, D2), lambda i: (i, 0)),
        pl.BlockSpec((NC, BR), lambda i: (0, i)),
        pl.BlockSpec((1, N_CLASSES), lambda i: (0, 0)),
    ],
    out_specs=pl.BlockSpec((BR, N_CLASSES), lambda i: (i, 0)),
    out_shape=jax.ShapeDtypeStruct((N_NODES, N_CLASSES), jnp.float32),
)


def kernel(x, edge_index, W1, b1, W2, b2):
    dst = edge_index[1].astype(jnp.int32)
    (src_rows,) = lax.optimization_barrier((edge_index[0],))
    src = src_rows.astype(jnp.int32)
    w2p = jnp.zeros((HIDDEN, D2), jnp.float32).at[:, :N_CLASSES].set(W2)
    b1r = b1.reshape(1, HIDDEN)
    b2r = b2.reshape(1, N_CLASSES)

    degp = _deg_kernel(dst)
    degp_col = degp.reshape(NC, NPAD)
    ht1 = _tca(x, W1, degp_col)
    y1p = _spmm128(src, dst, ht1)
    ht2 = _tcb(y1p, ht1, degp_col, b1r, w2p)
    y2p = _spmm2(src, dst, ht2)
    out = _tcc(y2p, ht2, degp_col, b2r)
    return out


# speedup vs baseline: 1.1193x; 1.1193x over previous
import functools

import jax
import jax.numpy as jnp
from jax import lax
from jax.experimental import pallas as pl
from jax.experimental.pallas import tpu as pltpu
from jax.experimental.pallas import tpu_sc as plsc

N_NODES = 10000
N_EDGES = 320000
D_FEAT = 128
HIDDEN = 128
N_CLASSES = 3
D2 = 128

NC, NS, L = 2, 16, 16
NW = NC * NS
NPAD = 10240
CHUNK = 80
NCHUNKS = N_EDGES // CHUNK
CPW = -(-NCHUNKS // NW)
ROWS_PT = NPAD // NS
BPT = ROWS_PT // CHUNK
EPW = N_EDGES // NW
TPW = EPW // CHUNK
SCHUNK = 80
STPW = EPW // SCHUNK
SNB = 3
SNG = STPW // SNB
SNREM = STPW - SNG * SNB
SBPT = ROWS_PT // SCHUNK

_MESH = plsc.VectorSubcoreMesh(core_axis_name="c", subcore_axis_name="s")


def _zero_buf(buf, d, n=None):
    n = CHUNK if n is None else n
    zv = jnp.zeros((L,), jnp.float32)
    if d == 0:
        def body(i, _):
            buf[pl.ds(i * L, L)] = zv
            return 0
        lax.fori_loop(0, n // L, body, 0)
    else:
        def body(i, _):
            for j in range(d // L):
                buf[i, pl.ds(j * L, L)] = zv
            return 0
        lax.fori_loop(0, n, body, 0)


DNB = 4
DEPW = N_EDGES // NW // CHUNK
DNG = DEPW // DNB
DNREM = DEPW - DNG * DNB


@functools.partial(
    pl.kernel,
    out_type=jax.ShapeDtypeStruct((NC * NPAD,), jnp.float32),
    mesh=_MESH,
    scratch_types=[
        pltpu.VMEM((DNB, CHUNK), jnp.int32),
        pltpu.VMEM((CHUNK,), jnp.float32),
        pltpu.VMEM((ROWS_PT,), jnp.float32),
        pltpu.VMEM_SHARED((NPAD,), jnp.float32),
        pltpu.SemaphoreType.DMA,
        pltpu.SemaphoreType.DMA,
        pltpu.SemaphoreType.DMA,
        pltpu.SemaphoreType.DMA,
        pltpu.SemaphoreType.DMA,
        pltpu.SemaphoreType.DMA,
        pltpu.SemaphoreType.DMA,
        pltpu.SemaphoreType.DMA,
    ],
)
def _deg_kernel(dst_hbm, out_hbm, didx, buf, wb, acc,
                si0, si1, si2, si3, ss0, ss1, ss2, ss3):
    c = lax.axis_index("c")
    s = lax.axis_index("s")
    wid = s * NC + c
    base = wid * EPW
    sem_i = [si0, si1, si2, si3]
    sem_s = [ss0, ss1, ss2, ss3]
    _zero_buf(wb, 0, ROWS_PT)
    pltpu.sync_copy(wb, acc.at[pl.ds(s * ROWS_PT, ROWS_PT)])
    plsc.subcore_barrier()
    ov = jnp.ones((L,), jnp.float32)

    def fill(i, _):
        buf[pl.ds(i * L, L)] = ov
        return 0
    lax.fori_loop(0, CHUNK // L, fill, 0)

    def outer(i, _):
        ihandles = []
        for b in range(DNB):
            off = base + (i * DNB + b) * CHUNK

            @pl.when(i > 0)
            def _():
                pltpu.make_async_copy(dst_hbm.at[pl.ds(0, CHUNK)],
                                      didx.at[b], sem_s[b]).wait()
            ihandles.append(
                pltpu.async_copy(dst_hbm.at[pl.ds(off, CHUNK)],
                                 didx.at[b], sem_i[b]))
        for b in range(DNB):
            ihandles[b].wait()
            pltpu.async_copy(buf, acc.at[didx.at[b]], sem_s[b], add=True)
        return 0
    lax.fori_loop(0, DNG, outer, 0)
    for b in range(DNB):
        pltpu.make_async_copy(dst_hbm.at[pl.ds(0, CHUNK)], didx.at[b],
                              sem_s[b]).wait()
    for r in range(DNREM):
        off = base + (DNG * DNB + r) * CHUNK
        pltpu.sync_copy(dst_hbm.at[pl.ds(off, CHUNK)], didx.at[0])
        pltpu.sync_copy(buf, acc.at[didx.at[0]], add=True)
    plsc.subcore_barrier()
    pltpu.sync_copy(acc.at[pl.ds(s * ROWS_PT, ROWS_PT)], wb)
    pltpu.sync_copy(wb, out_hbm.at[pl.ds(c * NPAD + s * ROWS_PT, ROWS_PT)])


def _make_spmm(d):
    @functools.partial(
        pl.kernel,
        out_type=jax.ShapeDtypeStruct((NC, NPAD, d), jnp.float32),
        mesh=_MESH,
        scratch_types=[
            pltpu.VMEM((EPW,), jnp.int32),
            pltpu.VMEM((SNB, SCHUNK), jnp.int32),
            pltpu.VMEM((SNB, SCHUNK, d), jnp.float32),
            pltpu.VMEM_SHARED((NPAD, d), jnp.float32),
            pltpu.SemaphoreType.DMA,
            pltpu.SemaphoreType.DMA,
            pltpu.SemaphoreType.DMA,
            pltpu.SemaphoreType.DMA,
            pltpu.SemaphoreType.DMA,
            pltpu.SemaphoreType.DMA,
            pltpu.SemaphoreType.DMA,
            pltpu.SemaphoreType.DMA,
            pltpu.SemaphoreType.DMA,
        ],
    )
    def spmm(src_hbm, dst_hbm, h_hbm, out_hbm, sidx_all, didx, rows,
             acc, si0, si1, si2, sg0, sg1, sg2, ss0, ss1, ss2):
        c = lax.axis_index("c")
        s = lax.axis_index("s")
        wid = s * NC + c
        base = wid * EPW
        sem_i = [si0, si1, si2]
        sem_g = [sg0, sg1, sg2]
        sem_s = [ss0, ss1, ss2]

        pre = pltpu.async_copy(src_hbm.at[pl.ds(base, EPW)], sidx_all, si0)

        zv = jnp.zeros((L,), jnp.float32)

        def zbody(i, _):
            for j in range(d // L):
                rows[0, i, pl.ds(j * L, L)] = zv
            return 0
        lax.fori_loop(0, SCHUNK, zbody, 0)
        for k in range(SBPT):
            pltpu.sync_copy(rows.at[0],
                            acc.at[pl.ds(s * ROWS_PT + k * SCHUNK, SCHUNK)])
        pre.wait()
        plsc.subcore_barrier()

        def outer(i, _):
            ihandles = []
            ghandles = []
            for b in range(SNB):
                t = i * SNB + b

                @pl.when(i > 0)
                def _():
                    pltpu.make_async_copy(h_hbm.at[pl.ds(0, SCHUNK)],
                                          rows.at[b], sem_s[b]).wait()
                ghandles.append(
                    pltpu.async_copy(
                        h_hbm.at[sidx_all.at[pl.ds(t * SCHUNK, SCHUNK)]],
                        rows.at[b], sem_g[b]))
                ihandles.append(
                    pltpu.async_copy(dst_hbm.at[pl.ds(base + t * SCHUNK, SCHUNK)],
                                     didx.at[b], sem_i[b]))
            for b in range(SNB):
                ghandles[b].wait()
                ihandles[b].wait()
                pltpu.async_copy(rows.at[b], acc.at[didx.at[b]], sem_s[b],
                                 add=True)
            return 0
        lax.fori_loop(0, SNG, outer, 0)
        for b in range(SNB):
            pltpu.make_async_copy(h_hbm.at[pl.ds(0, SCHUNK)], rows.at[b],
                                  sem_s[b]).wait()
        for r in range(SNREM):
            t = SNG * SNB + r
            pltpu.sync_copy(dst_hbm.at[pl.ds(base + t * SCHUNK, SCHUNK)],
                            didx.at[0])
            pltpu.async_copy(
                h_hbm.at[sidx_all.at[pl.ds(t * SCHUNK, SCHUNK)]],
                rows.at[0], sg0).wait()
            pltpu.sync_copy(rows.at[0], acc.at[didx.at[0]], add=True)
        plsc.subcore_barrier()
        for k in range(SBPT):
            r0 = s * ROWS_PT + k * SCHUNK
            pltpu.sync_copy(acc.at[pl.ds(r0, SCHUNK)], rows.at[0])
            pltpu.sync_copy(rows.at[0], out_hbm.at[c, pl.ds(r0, SCHUNK)])

    return spmm


_spmm128 = _make_spmm(HIDDEN)
_spmm2 = _make_spmm(D2)

BR = 640
GRID = NPAD // BR


def _dis(degp_blk):
    d = lax.rsqrt(degp_blk[0] + degp_blk[1] + 1.0)
    return d[:, None]


def _tca_body(x_ref, w1_ref, degp_ref, ht_ref):
    dis = _dis(degp_ref[...])
    h = jnp.dot(x_ref[...], w1_ref[...], preferred_element_type=jnp.float32)
    ht_ref[...] = h * dis


_tca = pl.pallas_call(
    _tca_body,
    grid=(GRID,),
    in_specs=[
        pl.BlockSpec((BR, D_FEAT), lambda i: (i, 0)),
        pl.BlockSpec((D_FEAT, HIDDEN), lambda i: (0, 0)),
        pl.BlockSpec((NC, BR), lambda i: (0, i)),
    ],
    out_specs=pl.BlockSpec((BR, HIDDEN), lambda i: (i, 0)),
    out_shape=jax.ShapeDtypeStruct((N_NODES, HIDDEN), jnp.float32),
)


def _tcb_body(y1_ref, ht1_ref, degp_ref, b1_ref, w2_ref, ht2_ref):
    dis = _dis(degp_ref[...])
    y = y1_ref[0] + y1_ref[1] + ht1_ref[...]
    o1 = y * dis + b1_ref[...]
    r = jnp.maximum(o1, 0.0)
    h2 = jnp.dot(r, w2_ref[...], preferred_element_type=jnp.float32)
    ht2_ref[...] = h2 * dis


_tcb = pl.pallas_call(
    _tcb_body,
    grid=(GRID,),
    in_specs=[
        pl.BlockSpec((NC, BR, HIDDEN), lambda i: (0, i, 0)),
        pl.BlockSpec((BR, HIDDEN), lambda i: (i, 0)),
        pl.BlockSpec((NC, BR), lambda i: (0, i)),
        pl.BlockSpec((1, HIDDEN), lambda i: (0, 0)),
        pl.BlockSpec((HIDDEN, D2), lambda i: (0, 0)),
    ],
    out_specs=pl.BlockSpec((BR, D2), lambda i: (i, 0)),
    out_shape=jax.ShapeDtypeStruct((N_NODES, D2), jnp.float32),
)


def _tcc_body(y2_ref, ht2_ref, degp_ref, b2_ref, out_ref):
    dis = _dis(degp_ref[...])
    y = y2_ref[0] + y2_ref[1] + ht2_ref[...]
    o = y * dis
    out_ref[...] = o[:, :N_CLASSES] + b2_ref[...]


_tcc = pl.pallas_call(
    _tcc_body,
    grid=(GRID,),
    in_specs=[
        pl.BlockSpec((NC, BR, D2), lambda i: (0, i, 0)),
        pl.BlockSpec((BR, D2), lambda i: (i, 0)),
        pl.BlockSpec((NC, BR), lambda i: (0, i)),
        pl.BlockSpec((1, N_CLASSES), lambda i: (0, 0)),
    ],
    out_specs=pl.BlockSpec((BR, N_CLASSES), lambda i: (i, 0)),
    out_shape=jax.ShapeDtypeStruct((N_NODES, N_CLASSES), jnp.float32),
)


def kernel(x, edge_index, W1, b1, W2, b2):
    dst = edge_index[1].astype(jnp.int32)
    (src_rows,) = lax.optimization_barrier((edge_index[0],))
    src = src_rows.astype(jnp.int32)
    w2p = jnp.zeros((HIDDEN, D2), jnp.float32).at[:, :N_CLASSES].set(W2)
    b1r = b1.reshape(1, HIDDEN)
    b2r = b2.reshape(1, N_CLASSES)

    degp = _deg_kernel(dst)
    degp_col = degp.reshape(NC, NPAD)
    ht1 = _tca(x, W1, degp_col)
    y1p = _spmm128(src, dst, ht1)
    ht2 = _tcb(y1p, ht1, degp_col, b1r, w2p)
    y2p = _spmm2(src, dst, ht2)
    out = _tcc(y2p, ht2, degp_col, b2r)
    return out


# speedup vs baseline: 1.1267x; 1.0066x over previous
import functools

import jax
import jax.numpy as jnp
from jax import lax
from jax.experimental import pallas as pl
from jax.experimental.pallas import tpu as pltpu
from jax.experimental.pallas import tpu_sc as plsc

N_NODES = 10000
N_EDGES = 320000
D_FEAT = 128
HIDDEN = 128
N_CLASSES = 3
D2 = 128

NC, NS, L = 2, 16, 16
NW = NC * NS
NPAD = 10240
CHUNK = 80
NCHUNKS = N_EDGES // CHUNK
CPW = -(-NCHUNKS // NW)
ROWS_PT = NPAD // NS
BPT = ROWS_PT // CHUNK
EPW = N_EDGES // NW
TPW = EPW // CHUNK
SCHUNK = 80
STPW = EPW // SCHUNK
SNB = 3
SNG = STPW // SNB
SNREM = STPW - SNG * SNB
SBPT = ROWS_PT // SCHUNK

_MESH = plsc.VectorSubcoreMesh(core_axis_name="c", subcore_axis_name="s")


def _zero_buf(buf, d, n=None):
    n = CHUNK if n is None else n
    zv = jnp.zeros((L,), jnp.float32)
    if d == 0:
        def body(i, _):
            buf[pl.ds(i * L, L)] = zv
            return 0
        lax.fori_loop(0, n // L, body, 0)
    else:
        def body(i, _):
            for j in range(d // L):
                buf[i, pl.ds(j * L, L)] = zv
            return 0
        lax.fori_loop(0, n, body, 0)


DNB = 5
DEPW = N_EDGES // NW // CHUNK
DNG = DEPW // DNB
DNREM = DEPW - DNG * DNB


@functools.partial(
    pl.kernel,
    out_type=jax.ShapeDtypeStruct((NC * NPAD,), jnp.float32),
    mesh=_MESH,
    scratch_types=[
        pltpu.VMEM((DNB, CHUNK), jnp.int32),
        pltpu.VMEM((CHUNK,), jnp.float32),
        pltpu.VMEM((ROWS_PT,), jnp.float32),
        pltpu.VMEM_SHARED((NPAD,), jnp.float32),
        pltpu.SemaphoreType.DMA,
        pltpu.SemaphoreType.DMA,
        pltpu.SemaphoreType.DMA,
        pltpu.SemaphoreType.DMA,
        pltpu.SemaphoreType.DMA,
        pltpu.SemaphoreType.DMA,
        pltpu.SemaphoreType.DMA,
        pltpu.SemaphoreType.DMA,
        pltpu.SemaphoreType.DMA,
        pltpu.SemaphoreType.DMA,
    ],
)
def _deg_kernel(dst_hbm, out_hbm, didx, buf, wb, acc,
                si0, si1, si2, si3, si4, ss0, ss1, ss2, ss3, ss4):
    c = lax.axis_index("c")
    s = lax.axis_index("s")
    wid = s * NC + c
    base = wid * EPW
    sem_i = [si0, si1, si2, si3, si4]
    sem_s = [ss0, ss1, ss2, ss3, ss4]
    _zero_buf(wb, 0, ROWS_PT)
    pltpu.sync_copy(wb, acc.at[pl.ds(s * ROWS_PT, ROWS_PT)])
    plsc.subcore_barrier()
    ov = jnp.ones((L,), jnp.float32)

    def fill(i, _):
        buf[pl.ds(i * L, L)] = ov
        return 0
    lax.fori_loop(0, CHUNK // L, fill, 0)

    def outer(i, _):
        ihandles = []
        for b in range(DNB):
            off = base + (i * DNB + b) * CHUNK

            @pl.when(i > 0)
            def _():
                pltpu.make_async_copy(dst_hbm.at[pl.ds(0, CHUNK)],
                                      didx.at[b], sem_s[b]).wait()
            ihandles.append(
                pltpu.async_copy(dst_hbm.at[pl.ds(off, CHUNK)],
                                 didx.at[b], sem_i[b]))
        for b in range(DNB):
            ihandles[b].wait()
            pltpu.async_copy(buf, acc.at[didx.at[b]], sem_s[b], add=True)
        return 0
    lax.fori_loop(0, DNG, outer, 0)
    for b in range(DNB):
        pltpu.make_async_copy(dst_hbm.at[pl.ds(0, CHUNK)], didx.at[b],
                              sem_s[b]).wait()
    for r in range(DNREM):
        off = base + (DNG * DNB + r) * CHUNK
        pltpu.sync_copy(dst_hbm.at[pl.ds(off, CHUNK)], didx.at[0])
        pltpu.sync_copy(buf, acc.at[didx.at[0]], add=True)
    plsc.subcore_barrier()
    pltpu.sync_copy(acc.at[pl.ds(s * ROWS_PT, ROWS_PT)], wb)
    pltpu.sync_copy(wb, out_hbm.at[pl.ds(c * NPAD + s * ROWS_PT, ROWS_PT)])


def _make_spmm(d):
    @functools.partial(
        pl.kernel,
        out_type=jax.ShapeDtypeStruct((NC, NPAD, d), jnp.float32),
        mesh=_MESH,
        scratch_types=[
            pltpu.VMEM((EPW,), jnp.int32),
            pltpu.VMEM((SNB, SCHUNK), jnp.int32),
            pltpu.VMEM((SNB, SCHUNK, d), jnp.float32),
            pltpu.VMEM_SHARED((NPAD, d), jnp.float32),
            pltpu.SemaphoreType.DMA,
            pltpu.SemaphoreType.DMA,
            pltpu.SemaphoreType.DMA,
            pltpu.SemaphoreType.DMA,
            pltpu.SemaphoreType.DMA,
            pltpu.SemaphoreType.DMA,
            pltpu.SemaphoreType.DMA,
            pltpu.SemaphoreType.DMA,
            pltpu.SemaphoreType.DMA,
        ],
    )
    def spmm(src_hbm, dst_hbm, h_hbm, out_hbm, sidx_all, didx, rows,
             acc, si0, si1, si2, sg0, sg1, sg2, ss0, ss1, ss2):
        c = lax.axis_index("c")
        s = lax.axis_index("s")
        wid = s * NC + c
        base = wid * EPW
        sem_i = [si0, si1, si2]
        sem_g = [sg0, sg1, sg2]
        sem_s = [ss0, ss1, ss2]

        pre = pltpu.async_copy(src_hbm.at[pl.ds(base, EPW)], sidx_all, si0)

        zv = jnp.zeros((L,), jnp.float32)

        def zbody(i, _):
            for j in range(d // L):
                rows[0, i, pl.ds(j * L, L)] = zv
            return 0
        lax.fori_loop(0, SCHUNK, zbody, 0)
        for k in range(SBPT):
            pltpu.sync_copy(rows.at[0],
                            acc.at[pl.ds(s * ROWS_PT + k * SCHUNK, SCHUNK)])
        pre.wait()
        plsc.subcore_barrier()

        def outer(i, _):
            ihandles = []
            ghandles = []
            for b in range(SNB):
                t = i * SNB + b

                @pl.when(i > 0)
                def _():
                    pltpu.make_async_copy(h_hbm.at[pl.ds(0, SCHUNK)],
                                          rows.at[b], sem_s[b]).wait()
                ghandles.append(
                    pltpu.async_copy(
                        h_hbm.at[sidx_all.at[pl.ds(t * SCHUNK, SCHUNK)]],
                        rows.at[b], sem_g[b]))
                ihandles.append(
                    pltpu.async_copy(dst_hbm.at[pl.ds(base + t * SCHUNK, SCHUNK)],
                                     didx.at[b], sem_i[b]))
            for b in range(SNB):
                ghandles[b].wait()
                ihandles[b].wait()
                pltpu.async_copy(rows.at[b], acc.at[didx.at[b]], sem_s[b],
                                 add=True)
            return 0
        lax.fori_loop(0, SNG, outer, 0)
        for b in range(SNB):
            pltpu.make_async_copy(h_hbm.at[pl.ds(0, SCHUNK)], rows.at[b],
                                  sem_s[b]).wait()
        for r in range(SNREM):
            t = SNG * SNB + r
            pltpu.sync_copy(dst_hbm.at[pl.ds(base + t * SCHUNK, SCHUNK)],
                            didx.at[0])
            pltpu.async_copy(
                h_hbm.at[sidx_all.at[pl.ds(t * SCHUNK, SCHUNK)]],
                rows.at[0], sg0).wait()
            pltpu.sync_copy(rows.at[0], acc.at[didx.at[0]], add=True)
        plsc.subcore_barrier()
        for k in range(SBPT):
            r0 = s * ROWS_PT + k * SCHUNK
            pltpu.sync_copy(acc.at[pl.ds(r0, SCHUNK)], rows.at[0])
            pltpu.sync_copy(rows.at[0], out_hbm.at[c, pl.ds(r0, SCHUNK)])

    return spmm


_spmm128 = _make_spmm(HIDDEN)
_spmm2 = _make_spmm(D2)

BR = 640
GRID = NPAD // BR


def _dis(degp_blk):
    d = lax.rsqrt(degp_blk[0] + degp_blk[1] + 1.0)
    return d[:, None]


def _tca_body(x_ref, w1_ref, degp_ref, ht_ref):
    dis = _dis(degp_ref[...])
    h = jnp.dot(x_ref[...], w1_ref[...], preferred_element_type=jnp.float32)
    ht_ref[...] = h * dis


_tca = pl.pallas_call(
    _tca_body,
    grid=(GRID,),
    in_specs=[
        pl.BlockSpec((BR, D_FEAT), lambda i: (i, 0)),
        pl.BlockSpec((D_FEAT, HIDDEN), lambda i: (0, 0)),
        pl.BlockSpec((NC, BR), lambda i: (0, i)),
    ],
    out_specs=pl.BlockSpec((BR, HIDDEN), lambda i: (i, 0)),
    out_shape=jax.ShapeDtypeStruct((N_NODES, HIDDEN), jnp.float32),
)


def _tcb_body(y1_ref, ht1_ref, degp_ref, b1_ref, w2_ref, ht2_ref):
    dis = _dis(degp_ref[...])
    y = y1_ref[0] + y1_ref[1] + ht1_ref[...]
    o1 = y * dis + b1_ref[...]
    r = jnp.maximum(o1, 0.0)
    h2 = jnp.dot(r, w2_ref[...], preferred_element_type=jnp.float32)
    ht2_ref[...] = h2 * dis


_tcb = pl.pallas_call(
    _tcb_body,
    grid=(GRID,),
    in_specs=[
        pl.BlockSpec((NC, BR, HIDDEN), lambda i: (0, i, 0)),
        pl.BlockSpec((BR, HIDDEN), lambda i: (i, 0)),
        pl.BlockSpec((NC, BR), lambda i: (0, i)),
        pl.BlockSpec((1, HIDDEN), lambda i: (0, 0)),
        pl.BlockSpec((HIDDEN, D2), lambda i: (0, 0)),
    ],
    out_specs=pl.BlockSpec((BR, D2), lambda i: (i, 0)),
    out_shape=jax.ShapeDtypeStruct((N_NODES, D2), jnp.float32),
)


def _tcc_body(y2_ref, ht2_ref, degp_ref, b2_ref, out_ref):
    dis = _dis(degp_ref[...])
    y = y2_ref[0] + y2_ref[1] + ht2_ref[...]
    o = y * dis
    out_ref[...] = o[:, :N_CLASSES] + b2_ref[...]


_tcc = pl.pallas_call(
    _tcc_body,
    grid=(GRID,),
    in_specs=[
        pl.BlockSpec((NC, BR, D2), lambda i: (0, i, 0)),
        pl.BlockSpec((BR, D2), lambda i: (i, 0)),
        pl.BlockSpec((NC, BR), lambda i: (0, i)),
        pl.BlockSpec((1, N_CLASSES), lambda i: (0, 0)),
    ],
    out_specs=pl.BlockSpec((BR, N_CLASSES), lambda i: (i, 0)),
    out_shape=jax.ShapeDtypeStruct((N_NODES, N_CLASSES), jnp.float32),
)


def kernel(x, edge_index, W1, b1, W2, b2):
    dst = edge_index[1].astype(jnp.int32)
    (src_rows,) = lax.optimization_barrier((edge_index[0],))
    src = src_rows.astype(jnp.int32)
    w2p = jnp.zeros((HIDDEN, D2), jnp.float32).at[:, :N_CLASSES].set(W2)
    b1r = b1.reshape(1, HIDDEN)
    b2r = b2.reshape(1, N_CLASSES)

    degp = _deg_kernel(dst)
    degp_col = degp.reshape(NC, NPAD)
    ht1 = _tca(x, W1, degp_col)
    y1p = _spmm128(src, dst, ht1)
    ht2 = _tcb(y1p, ht1, degp_col, b1r, w2p)
    y2p = _spmm2(src, dst, ht2)
    out = _tcc(y2p, ht2, degp_col, b2r)
    return out
